# Initial kernel scaffold; baseline (speedup 1.0000x reference)
import jax
import jax.numpy as jnp
from jax.experimental import pallas as pl


def kernel(g, h, Wd0, bd0, Wd1, bd1, Wb, bb, Wu0, bu0, Wu1, bu1, p0, pb0, p1, pb1):
    raise NotImplementedError("write your pallas kernel here")


# speedup vs baseline: 1.5945x; 1.5945x over previous
import jax
import jax.numpy as jnp
from jax.experimental import pallas as pl

_N = 2048
_D = 256
_KN0 = 1638
_KN1 = 982
_BM = 256
_BT = 512

_F32 = jnp.float32
_BF16 = jnp.bfloat16


def _sig(x):
    return jax.nn.sigmoid(x)


def _k_down0(g_ref, h_ref, w_ref, b_ref, p_ref, pb_ref,
             h0_ref, deg_ref, w0_ref, ug_ref):
    gblk = g_ref[...]
    deg = jnp.maximum(jnp.sum(gblk, axis=1, keepdims=True), 1e-12)
    agg = jnp.dot(gblk, h_ref[...], preferred_element_type=_F32) / deg
    h0 = jax.nn.relu(jnp.dot(agg, w_ref[...], preferred_element_type=_F32)
                     + b_ref[...])
    h0_ref[...] = h0
    deg_ref[...] = deg
    w0_ref[...] = jnp.sum(h0 * p_ref[...], axis=1, keepdims=True) + pb_ref[...]
    ug_ref[...] = (gblk > 0).astype(_BF16)


def _k_rank0(w0c_ref, w0r_ref, h0_ref, x0_ref, m0_ref, r0_ref):
    rb = pl.program_id(0)
    sc = _sig(w0c_ref[...])
    sr = _sig(w0r_ref[...])
    gt = jnp.sum((sr > sc).astype(_F32), axis=1, keepdims=True)
    jglob = jax.lax.broadcasted_iota(jnp.int32, (_BM, _N), 1)
    iglob = jax.lax.broadcasted_iota(jnp.int32, (_BM, _N), 0) + rb * _BM
    eqb = jnp.sum(((sr == sc) & (jglob < iglob)).astype(_F32),
                  axis=1, keepdims=True)
    rank = gt + eqb
    m0 = (rank < float(_KN0)).astype(_F32)
    r0_ref[...] = rank
    m0_ref[...] = m0
    x0_ref[...] = h0_ref[...] * (m0 * sc)


def _k_twohop(a_ref, b_ref, t_ref):
    cnt = jnp.dot(a_ref[...], b_ref[...], preferred_element_type=_F32)
    t_ref[...] = (cnt > 0).astype(_BF16)


def _k_twohop_masked(a_ref, b_ref, m0r_ref, m0c_ref, u_ref):
    m0r = m0r_ref[...].astype(_BF16)
    m0c = m0c_ref[...].astype(_BF16)
    cnt = jnp.dot(a_ref[...] * m0r, b_ref[...] * m0c,
                  preferred_element_type=_F32)
    u_ref[...] = (cnt > 0).astype(_BF16)


def _k_gcn1(t_ref, x0_ref, m0r_ref, wd1_ref, bd1_ref, p1_ref, pb1_ref,
            h1_ref, w1_ref, deg1_ref):
    tf = t_ref[...].astype(_F32)
    deg1 = jnp.maximum(jnp.sum(tf * m0r_ref[...], axis=1, keepdims=True),
                       1e-12)
    agg = jnp.dot(tf, x0_ref[...], preferred_element_type=_F32) / deg1
    h1 = jax.nn.relu(jnp.dot(agg, wd1_ref[...], preferred_element_type=_F32)
                     + bd1_ref[...])
    h1_ref[...] = h1
    deg1_ref[...] = deg1
    w1_ref[...] = jnp.sum(h1 * p1_ref[...], axis=1, keepdims=True) + pb1_ref[...]


def _k_rank1(w1c_ref, w1r_ref, m0c_ref, m0r_ref, r0c_ref, r0r_ref, h1_ref,
             x1_ref, m1_ref):
    sc = _sig(w1c_ref[...])
    sr = _sig(w1r_ref[...])
    m0r = m0r_ref[...]
    gt = jnp.sum((sr > sc).astype(_F32) * m0r, axis=1, keepdims=True)
    eqb = jnp.sum(((sr == sc) & (r0r_ref[...] < r0c_ref[...])).astype(_F32)
                  * m0r, axis=1, keepdims=True)
    m1 = (gt + eqb < float(_KN1)).astype(_F32) * m0c_ref[...]
    m1_ref[...] = m1
    x1_ref[...] = h1_ref[...] * (m1 * sc)


def _k_gcn2(u_ref, x1_ref, m1r_ref, m1c_ref, wb_ref, bb_ref, y2_ref):
    uf = u_ref[...].astype(_F32)
    deg2 = jnp.maximum(jnp.sum(uf * m1r_ref[...], axis=1, keepdims=True),
                       1e-12)
    agg = jnp.dot(uf, x1_ref[...], preferred_element_type=_F32) / deg2
    h2 = jax.nn.relu(jnp.dot(agg, wb_ref[...], preferred_element_type=_F32)
                     + bb_ref[...])
    y2_ref[...] = h2 * m1c_ref[...]


def _k_up1(t_ref, y2_ref, deg1_ref, h1_ref, m0c_ref, wu0_ref, bu0_ref, z_ref):
    tf = t_ref[...].astype(_F32)
    agg = jnp.dot(tf, y2_ref[...], preferred_element_type=_F32) / deg1_ref[...]
    hu = jax.nn.relu(jnp.dot(agg, wu0_ref[...], preferred_element_type=_F32)
                     + bu0_ref[...])
    z_ref[...] = (hu + h1_ref[...]) * m0c_ref[...]


def _k_up0(g_ref, z_ref, deg_ref, h0_ref, h_ref, wu1_ref, bu1_ref,
           hs1_ref, hs2_ref):
    agg = jnp.dot(g_ref[...], z_ref[...], preferred_element_type=_F32) \
        / deg_ref[...]
    hu = jax.nn.relu(jnp.dot(agg, wu1_ref[...], preferred_element_type=_F32)
                     + bu1_ref[...])
    hs1 = hu + h0_ref[...]
    hs1_ref[...] = hs1
    hs2_ref[...] = hs1 + h_ref[...]


def _k_compact(r0r_ref, z_ref, out_ref):
    rb = pl.program_id(0)
    rows = (jax.lax.broadcasted_iota(jnp.int32, (_BM, _N), 0)
            + rb * _BM).astype(_F32)
    onehot = (r0r_ref[...] == rows).astype(_F32)
    out_ref[...] = jnp.dot(onehot, z_ref[...], preferred_element_type=_F32)


def _full(shape):
    n = len(shape)
    return pl.BlockSpec(shape, lambda i: (0,) * n)


def _rows(width):
    return pl.BlockSpec((_BM, width), lambda i: (i, 0))


def kernel(g, h, Wd0, bd0, Wd1, bd1, Wb, bb, Wu0, bu0, Wu1, bu1,
           p0, pb0, p1, pb1):
    grid = (_N // _BM,)
    row_shape = lambda w, d=_F32: jax.ShapeDtypeStruct((_N, w), d)

    h0, deg, w0, ug = pl.pallas_call(
        _k_down0,
        grid=grid,
        in_specs=[_rows(_N), _full((_N, _D)), _full((_D, _D)),
                  _full((1, _D)), _full((1, _D)), _full((1, 1))],
        out_specs=[_rows(_D), _rows(1), _rows(1), _rows(_N)],
        out_shape=[row_shape(_D), row_shape(1), row_shape(1),
                   jax.ShapeDtypeStruct((_N, _N), _BF16)],
    )(g, h, Wd0, bd0.reshape(1, _D), p0.reshape(1, _D), pb0.reshape(1, 1))

    x0, m0, r0 = pl.pallas_call(
        _k_rank0,
        grid=grid,
        in_specs=[_rows(1), _full((1, _N)), _rows(_D)],
        out_specs=[_rows(_D), _rows(1), _rows(1)],
        out_shape=[row_shape(_D), row_shape(1), row_shape(1)],
    )(w0, w0.reshape(1, _N), h0)
    m0r = m0.reshape(1, _N)
    r0r = r0.reshape(1, _N)

    tgrid = (_N // _BT, _N // _BT)
    T = pl.pallas_call(
        _k_twohop,
        grid=tgrid,
        in_specs=[pl.BlockSpec((_BT, _N), lambda i, j: (i, 0)),
                  pl.BlockSpec((_N, _BT), lambda i, j: (0, j))],
        out_specs=pl.BlockSpec((_BT, _BT), lambda i, j: (i, j)),
        out_shape=jax.ShapeDtypeStruct((_N, _N), _BF16),
    )(ug, ug)

    h1, w1, deg1 = pl.pallas_call(
        _k_gcn1,
        grid=grid,
        in_specs=[_rows(_N), _full((_N, _D)), _full((1, _N)),
                  _full((_D, _D)), _full((1, _D)), _full((1, _D)),
                  _full((1, 1))],
        out_specs=[_rows(_D), _rows(1), _rows(1)],
        out_shape=[row_shape(_D), row_shape(1), row_shape(1)],
    )(T, x0, m0r, Wd1, bd1.reshape(1, _D), p1.reshape(1, _D),
      pb1.reshape(1, 1))

    x1, m1 = pl.pallas_call(
        _k_rank1,
        grid=grid,
        in_specs=[_rows(1), _full((1, _N)), _rows(1), _full((1, _N)),
                  _rows(1), _full((1, _N)), _rows(_D)],
        out_specs=[_rows(_D), _rows(1)],
        out_shape=[row_shape(_D), row_shape(1)],
    )(w1, w1.reshape(1, _N), m0, m0r, r0, r0r, h1)
    m1r = m1.reshape(1, _N)

    U = pl.pallas_call(
        _k_twohop_masked,
        grid=tgrid,
        in_specs=[pl.BlockSpec((_BT, _N), lambda i, j: (i, 0)),
                  pl.BlockSpec((_N, _BT), lambda i, j: (0, j)),
                  pl.BlockSpec((1, _N), lambda i, j: (0, 0)),
                  pl.BlockSpec((_N, 1), lambda i, j: (0, 0))],
        out_specs=pl.BlockSpec((_BT, _BT), lambda i, j: (i, j)),
        out_shape=jax.ShapeDtypeStruct((_N, _N), _BF16),
    )(T, T, m0r, m0)

    y2 = pl.pallas_call(
        _k_gcn2,
        grid=grid,
        in_specs=[_rows(_N), _full((_N, _D)), _full((1, _N)), _rows(1),
                  _full((_D, _D)), _full((1, _D))],
        out_specs=_rows(_D),
        out_shape=row_shape(_D),
    )(U, x1, m1r, m1, Wb, bb.reshape(1, _D))

    z = pl.pallas_call(
        _k_up1,
        grid=grid,
        in_specs=[_rows(_N), _full((_N, _D)), _rows(1), _rows(_D), _rows(1),
                  _full((_D, _D)), _full((1, _D))],
        out_specs=_rows(_D),
        out_shape=row_shape(_D),
    )(T, y2, deg1, h1, m0, Wu0, bu0.reshape(1, _D))

    hs1, hs2 = pl.pallas_call(
        _k_up0,
        grid=grid,
        in_specs=[_rows(_N), _full((_N, _D)), _rows(1), _rows(_D), _rows(_D),
                  _full((_D, _D)), _full((1, _D))],
        out_specs=[_rows(_D), _rows(_D)],
        out_shape=[row_shape(_D), row_shape(_D)],
    )(g, z, deg, h0, h, Wu1, bu1.reshape(1, _D))

    out0_pad = pl.pallas_call(
        _k_compact,
        grid=grid,
        in_specs=[_full((1, _N)), _full((_N, _D))],
        out_specs=_rows(_D),
        out_shape=row_shape(_D),
    )(r0r, z)

    return (out0_pad[:_KN0], hs1, hs2)


# speedup vs baseline: 1.8285x; 1.1468x over previous
import jax
import jax.numpy as jnp
from jax.experimental import pallas as pl

_N = 2048
_D = 256
_KN0 = 1638
_KN1 = 982
_BM = 256
_BT = 1024

_F32 = jnp.float32
_BF16 = jnp.bfloat16
_I8 = jnp.int8
_I32 = jnp.int32


def _sig(x):
    return jax.nn.sigmoid(x)


def _k_down0(g_ref, h_ref, w_ref, b_ref, p_ref, pb_ref,
             h0_ref, deg_ref, w0_ref, ug_ref):
    gblk = g_ref[...]
    deg = jnp.maximum(jnp.sum(gblk, axis=1, keepdims=True), 1e-12)
    agg = jnp.dot(gblk, h_ref[...], preferred_element_type=_F32) / deg
    h0 = jax.nn.relu(jnp.dot(agg, w_ref[...], preferred_element_type=_F32)
                     + b_ref[...])
    h0_ref[...] = h0
    deg_ref[...] = deg
    w0_ref[...] = jnp.sum(h0 * p_ref[...], axis=1, keepdims=True) + pb_ref[...]
    ug_ref[...] = (gblk > 0).astype(_I8)


def _k_rank0(w0c_ref, w0r_ref, h0_ref, x0_ref, m0_ref, r0_ref):
    rb = pl.program_id(0)
    sc = _sig(w0c_ref[...])
    sr = _sig(w0r_ref[...])
    gt = jnp.sum((sr > sc).astype(_F32), axis=1, keepdims=True)
    jglob = jax.lax.broadcasted_iota(jnp.int32, (_BM, _N), 1)
    iglob = jax.lax.broadcasted_iota(jnp.int32, (_BM, _N), 0) + rb * _BM
    eqb = jnp.sum(((sr == sc) & (jglob < iglob)).astype(_F32),
                  axis=1, keepdims=True)
    rank = gt + eqb
    m0 = (rank < float(_KN0)).astype(_F32)
    r0_ref[...] = rank
    m0_ref[...] = m0
    x0_ref[...] = h0_ref[...] * (m0 * sc)


def _k_twohop(a_ref, b_ref, m0r_ref, t_ref, tm_ref):
    cnt = jnp.dot(a_ref[...], b_ref[...], preferred_element_type=_I32)
    pos = cnt > 0
    t_ref[...] = pos.astype(_I8)
    tm_ref[...] = (pos & (m0r_ref[...] > 0.0)).astype(_I8)


def _k_twohop_masked(a_ref, b_ref, u_ref):
    cnt = jnp.dot(a_ref[...], b_ref[...], preferred_element_type=_I32)
    u_ref[...] = (cnt > 0).astype(_I8)


def _k_gcn1(t_ref, x0_ref, m0r_ref, wd1_ref, bd1_ref, p1_ref, pb1_ref,
            h1_ref, w1_ref, deg1_ref):
    tf = t_ref[...].astype(_F32)
    deg1 = jnp.maximum(jnp.sum(tf * m0r_ref[...], axis=1, keepdims=True),
                       1e-12)
    agg = jnp.dot(tf, x0_ref[...], preferred_element_type=_F32) / deg1
    h1 = jax.nn.relu(jnp.dot(agg, wd1_ref[...], preferred_element_type=_F32)
                     + bd1_ref[...])
    h1_ref[...] = h1
    deg1_ref[...] = deg1
    w1_ref[...] = jnp.sum(h1 * p1_ref[...], axis=1, keepdims=True) + pb1_ref[...]


def _k_rank1(w1c_ref, w1r_ref, m0c_ref, m0r_ref, r0c_ref, r0r_ref, h1_ref,
             x1_ref, m1_ref):
    sc = _sig(w1c_ref[...])
    sr = _sig(w1r_ref[...])
    m0r = m0r_ref[...]
    gt = jnp.sum((sr > sc).astype(_F32) * m0r, axis=1, keepdims=True)
    eqb = jnp.sum(((sr == sc) & (r0r_ref[...] < r0c_ref[...])).astype(_F32)
                  * m0r, axis=1, keepdims=True)
    m1 = (gt + eqb < float(_KN1)).astype(_F32) * m0c_ref[...]
    m1_ref[...] = m1
    x1_ref[...] = (h1_ref[...] * (m1 * sc)).astype(_BF16)


def _k_gcn2(u_ref, x1_ref, m1r_ref, m1c_ref, wb_ref, bb_ref, y2_ref):
    uf = u_ref[...].astype(_F32)
    deg2 = jnp.maximum(jnp.sum(uf * m1r_ref[...], axis=1, keepdims=True),
                       1e-12)
    agg = jnp.dot(u_ref[...].astype(_BF16), x1_ref[...],
                  preferred_element_type=_F32) / deg2
    h2 = jax.nn.relu(jnp.dot(agg.astype(_BF16), wb_ref[...],
                             preferred_element_type=_F32) + bb_ref[...])
    y2_ref[...] = (h2 * m1c_ref[...]).astype(_BF16)


def _k_up1(t_ref, y2_ref, deg1_ref, h1_ref, m0c_ref, wu0_ref, bu0_ref, z_ref):
    agg = jnp.dot(t_ref[...].astype(_BF16), y2_ref[...],
                  preferred_element_type=_F32) / deg1_ref[...]
    hu = jax.nn.relu(jnp.dot(agg.astype(_BF16), wu0_ref[...],
                             preferred_element_type=_F32) + bu0_ref[...])
    z_ref[...] = (hu + h1_ref[...]) * m0c_ref[...]


def _k_up0(g_ref, z_ref, deg_ref, h0_ref, h_ref, wu1_ref, bu1_ref,
           r0r_ref, zf_ref, hs1_ref, hs2_ref, out_ref):
    rb = pl.program_id(0)
    agg = jnp.dot(g_ref[...].astype(_BF16), z_ref[...].astype(_BF16),
                  preferred_element_type=_F32) / deg_ref[...]
    hu = jax.nn.relu(jnp.dot(agg.astype(_BF16), wu1_ref[...],
                             preferred_element_type=_F32) + bu1_ref[...])
    hs1 = hu + h0_ref[...]
    hs1_ref[...] = hs1
    hs2_ref[...] = hs1 + h_ref[...]
    rows = (jax.lax.broadcasted_iota(jnp.int32, (_BM, _N), 0)
            + rb * _BM).astype(_F32)
    onehot = (r0r_ref[...] == rows).astype(_BF16)
    out_ref[...] = jnp.dot(onehot, zf_ref[...].astype(_BF16),
                           preferred_element_type=_F32)


def _full(shape):
    n = len(shape)
    return pl.BlockSpec(shape, lambda i: (0,) * n)


def _rows(width):
    return pl.BlockSpec((_BM, width), lambda i: (i, 0))


def kernel(g, h, Wd0, bd0, Wd1, bd1, Wb, bb, Wu0, bu0, Wu1, bu1,
           p0, pb0, p1, pb1):
    grid = (_N // _BM,)
    row_shape = lambda w, d=_F32: jax.ShapeDtypeStruct((_N, w), d)

    h0, deg, w0, ug = pl.pallas_call(
        _k_down0,
        grid=grid,
        in_specs=[_rows(_N), _full((_N, _D)), _full((_D, _D)),
                  _full((1, _D)), _full((1, _D)), _full((1, 1))],
        out_specs=[_rows(_D), _rows(1), _rows(1), _rows(_N)],
        out_shape=[row_shape(_D), row_shape(1), row_shape(1),
                   jax.ShapeDtypeStruct((_N, _N), _I8)],
    )(g, h, Wd0, bd0.reshape(1, _D), p0.reshape(1, _D), pb0.reshape(1, 1))

    x0, m0, r0 = pl.pallas_call(
        _k_rank0,
        grid=grid,
        in_specs=[_rows(1), _full((1, _N)), _rows(_D)],
        out_specs=[_rows(_D), _rows(1), _rows(1)],
        out_shape=[row_shape(_D), row_shape(1), row_shape(1)],
    )(w0, w0.reshape(1, _N), h0)
    m0r = m0.reshape(1, _N)
    r0r = r0.reshape(1, _N)

    tgrid = (_N // _BT, _N // _BT)
    T, Tm = pl.pallas_call(
        _k_twohop,
        grid=tgrid,
        in_specs=[pl.BlockSpec((_BT, _N), lambda i, j: (i, 0)),
                  pl.BlockSpec((_N, _BT), lambda i, j: (0, j)),
                  pl.BlockSpec((1, _BT), lambda i, j: (0, j))],
        out_specs=[pl.BlockSpec((_BT, _BT), lambda i, j: (i, j)),
                   pl.BlockSpec((_BT, _BT), lambda i, j: (i, j))],
        out_shape=[jax.ShapeDtypeStruct((_N, _N), _I8),
                   jax.ShapeDtypeStruct((_N, _N), _I8)],
    )(ug, ug, m0r)

    h1, w1, deg1 = pl.pallas_call(
        _k_gcn1,
        grid=grid,
        in_specs=[_rows(_N), _full((_N, _D)), _full((1, _N)),
                  _full((_D, _D)), _full((1, _D)), _full((1, _D)),
                  _full((1, 1))],
        out_specs=[_rows(_D), _rows(1), _rows(1)],
        out_shape=[row_shape(_D), row_shape(1), row_shape(1)],
    )(T, x0, m0r, Wd1, bd1.reshape(1, _D), p1.reshape(1, _D),
      pb1.reshape(1, 1))

    x1, m1 = pl.pallas_call(
        _k_rank1,
        grid=grid,
        in_specs=[_rows(1), _full((1, _N)), _rows(1), _full((1, _N)),
                  _rows(1), _full((1, _N)), _rows(_D)],
        out_specs=[_rows(_D), _rows(1)],
        out_shape=[jax.ShapeDtypeStruct((_N, _D), _BF16), row_shape(1)],
    )(w1, w1.reshape(1, _N), m0, m0r, r0, r0r, h1)
    m1r = m1.reshape(1, _N)

    U = pl.pallas_call(
        _k_twohop_masked,
        grid=tgrid,
        in_specs=[pl.BlockSpec((_BT, _N), lambda i, j: (i, 0)),
                  pl.BlockSpec((_N, _BT), lambda i, j: (0, j))],
        out_specs=pl.BlockSpec((_BT, _BT), lambda i, j: (i, j)),
        out_shape=jax.ShapeDtypeStruct((_N, _N), _I8),
    )(Tm, T)

    y2 = pl.pallas_call(
        _k_gcn2,
        grid=grid,
        in_specs=[_rows(_N), _full((_N, _D)), _full((1, _N)), _rows(1),
                  _full((_D, _D)), _full((1, _D))],
        out_specs=_rows(_D),
        out_shape=jax.ShapeDtypeStruct((_N, _D), _BF16),
    )(U, x1, m1r, m1, Wb.astype(_BF16), bb.reshape(1, _D))

    z = pl.pallas_call(
        _k_up1,
        grid=grid,
        in_specs=[_rows(_N), _full((_N, _D)), _rows(1), _rows(_D), _rows(1),
                  _full((_D, _D)), _full((1, _D))],
        out_specs=_rows(_D),
        out_shape=row_shape(_D),
    )(T, y2, deg1, h1, m0, Wu0.astype(_BF16), bu0.reshape(1, _D))

    hs1, hs2, out0_pad = pl.pallas_call(
        _k_up0,
        grid=grid,
        in_specs=[_rows(_N), _full((_N, _D)), _rows(1), _rows(_D), _rows(_D),
                  _full((_D, _D)), _full((1, _D)), _full((1, _N)),
                  _full((_N, _D))],
        out_specs=[_rows(_D), _rows(_D), _rows(_D)],
        out_shape=[row_shape(_D), row_shape(_D), row_shape(_D)],
    )(g, z, deg, h0, h, Wu1.astype(_BF16), bu1.reshape(1, _D), r0r, z)

    return (out0_pad[:_KN0], hs1, hs2)


# speedup vs baseline: 2.4696x; 1.3506x over previous
import jax
import jax.numpy as jnp
from jax.experimental import pallas as pl
from jax.experimental.pallas import tpu as pltpu

_N = 2048
_D = 256
_KN0 = 1638
_KN1 = 982
_BM = 256
_BT = 512

_F32 = jnp.float32
_BF16 = jnp.bfloat16

_DEG, _W0, _R0, _M0, _DEG1, _W1, _M1 = 0, 1, 2, 3, 4, 5, 6

_P0, _T1, _P1, _T2, _P2 = 0, 8, 9, 17, 18
_P3, _T3, _P4, _T4, _P5 = 22, 30, 31, 39, 40
_P6, _P7, _P8 = 44, 52, 60
_STEPS = 68


def _k_mono(g_ref, h_ref, wd0_ref, bd0_ref, wd1_ref, bd1_ref, wb_ref, bb_ref,
            wu0_ref, bu0_ref, wu1_ref, bu1_ref, p0_ref, pb0_ref, p1_ref,
            pb1_ref, out0_ref, hs1_ref, hs2_ref,
            ug_s, t_s, h0_s, x0_s, h1_s, x1_s, y2_s, z_s, cols_s, rows_s):
    i = pl.program_id(0)
    sig = jax.nn.sigmoid

    @pl.when(i < _T1)
    def _down0():
        b = i - _P0
        sl = pl.ds(b * _BM, _BM)
        gblk = g_ref[...]
        deg = jnp.maximum(jnp.sum(gblk, axis=1, keepdims=True), 1e-12)
        agg = jnp.dot(gblk, h_ref[...], preferred_element_type=_F32) / deg
        h0 = jax.nn.relu(jnp.dot(agg, wd0_ref[...],
                                 preferred_element_type=_F32) + bd0_ref[...])
        h0_s[sl, :] = h0
        cols_s[sl, _DEG:_DEG + 1] = deg
        cols_s[sl, _W0:_W0 + 1] = (jnp.sum(h0 * p0_ref[...], axis=1,
                                           keepdims=True) + pb0_ref[...])
        ug_s[sl, :] = (gblk > 0).astype(_BF16)

    @pl.when((i == _T1) | (i == _T2) | (i == _T3) | (i == _T4))
    def _transpose():
        rows_s[...] = jnp.transpose(cols_s[...])

    @pl.when((i >= _P1) & (i < _T2))
    def _rank0():
        b = i - _P1
        sl = pl.ds(b * _BM, _BM)
        sc = sig(cols_s[sl, _W0:_W0 + 1])
        sr = sig(rows_s[_W0:_W0 + 1, :])
        gt = jnp.sum((sr > sc).astype(_F32), axis=1, keepdims=True)
        jglob = jax.lax.broadcasted_iota(jnp.int32, (_BM, _N), 1)
        iglob = jax.lax.broadcasted_iota(jnp.int32, (_BM, _N), 0) + b * _BM
        eqb = jnp.sum(((sr == sc) & (jglob < iglob)).astype(_F32),
                      axis=1, keepdims=True)
        rank = gt + eqb
        m0 = (rank < float(_KN0)).astype(_F32)
        cols_s[sl, _R0:_R0 + 1] = rank
        cols_s[sl, _M0:_M0 + 1] = m0
        x0_s[sl, :] = h0_s[sl, :] * (m0 * sc)

    @pl.when((i >= _P2) & (i < _P3))
    def _twohop_t():
        b = i - _P2
        sl = pl.ds(b * _BT, _BT)
        cnt = jnp.dot(ug_s[sl, :], ug_s[...], preferred_element_type=_F32)
        t_s[sl, :] = (cnt > 0).astype(_BF16)

    @pl.when((i >= _P3) & (i < _T3))
    def _down1():
        b = i - _P3
        sl = pl.ds(b * _BM, _BM)
        tf = t_s[sl, :].astype(_F32)
        m0r = rows_s[_M0:_M0 + 1, :]
        deg1 = jnp.maximum(jnp.sum(tf * m0r, axis=1, keepdims=True), 1e-12)
        agg = jnp.dot(tf, x0_s[...], preferred_element_type=_F32) / deg1
        h1 = jax.nn.relu(jnp.dot(agg, wd1_ref[...],
                                 preferred_element_type=_F32) + bd1_ref[...])
        h1_s[sl, :] = h1
        cols_s[sl, _DEG1:_DEG1 + 1] = deg1
        cols_s[sl, _W1:_W1 + 1] = (jnp.sum(h1 * p1_ref[...], axis=1,
                                           keepdims=True) + pb1_ref[...])

    @pl.when((i >= _P4) & (i < _T4))
    def _rank1():
        b = i - _P4
        sl = pl.ds(b * _BM, _BM)
        sc = sig(cols_s[sl, _W1:_W1 + 1])
        sr = sig(rows_s[_W1:_W1 + 1, :])
        m0r = rows_s[_M0:_M0 + 1, :]
        r0c = cols_s[sl, _R0:_R0 + 1]
        r0r = rows_s[_R0:_R0 + 1, :]
        gt = jnp.sum((sr > sc).astype(_F32) * m0r, axis=1, keepdims=True)
        eqb = jnp.sum(((sr == sc) & (r0r < r0c)).astype(_F32) * m0r,
                      axis=1, keepdims=True)
        m1 = (gt + eqb < float(_KN1)).astype(_F32) * cols_s[sl, _M0:_M0 + 1]
        cols_s[sl, _M1:_M1 + 1] = m1
        x1_s[sl, :] = (h1_s[sl, :] * (m1 * sc)).astype(_BF16)

    @pl.when((i >= _P5) & (i < _P6))
    def _twohop_u():
        b = i - _P5
        sl = pl.ds(b * _BT, _BT)
        m0r = rows_s[_M0:_M0 + 1, :]
        a = (t_s[sl, :].astype(_F32) * m0r).astype(_BF16)
        cnt = jnp.dot(a, t_s[...], preferred_element_type=_F32)
        ug_s[sl, :] = (cnt > 0).astype(_BF16)

    @pl.when((i >= _P6) & (i < _P7))
    def _bottleneck():
        b = i - _P6
        sl = pl.ds(b * _BM, _BM)
        ub = ug_s[sl, :]
        m1r = rows_s[_M1:_M1 + 1, :]
        deg2 = jnp.maximum(jnp.sum(ub.astype(_F32) * m1r, axis=1,
                                   keepdims=True), 1e-12)
        agg = jnp.dot(ub, x1_s[...], preferred_element_type=_F32) / deg2
        h2 = jax.nn.relu(jnp.dot(agg.astype(_BF16),
                                 wb_ref[...].astype(_BF16),
                                 preferred_element_type=_F32) + bb_ref[...])
        y2_s[sl, :] = (h2 * cols_s[sl, _M1:_M1 + 1]).astype(_BF16)

    @pl.when((i >= _P7) & (i < _P8))
    def _up1():
        b = i - _P7
        sl = pl.ds(b * _BM, _BM)
        agg = (jnp.dot(t_s[sl, :], y2_s[...], preferred_element_type=_F32)
               / cols_s[sl, _DEG1:_DEG1 + 1])
        hu = jax.nn.relu(jnp.dot(agg.astype(_BF16),
                                 wu0_ref[...].astype(_BF16),
                                 preferred_element_type=_F32) + bu0_ref[...])
        z_s[sl, :] = ((hu + h1_s[sl, :])
                      * cols_s[sl, _M0:_M0 + 1]).astype(_BF16)

    @pl.when(i >= _P8)
    def _up0_compact():
        b = i - _P8
        sl = pl.ds(b * _BM, _BM)
        agg = (jnp.dot(g_ref[...].astype(_BF16), z_s[...],
                       preferred_element_type=_F32)
               / cols_s[sl, _DEG:_DEG + 1])
        hu = jax.nn.relu(jnp.dot(agg.astype(_BF16),
                                 wu1_ref[...].astype(_BF16),
                                 preferred_element_type=_F32) + bu1_ref[...])
        hs1 = hu + h0_s[sl, :]
        hs1_ref[sl, :] = hs1
        hs2_ref[sl, :] = hs1 + h_ref[sl, :]
        rows = (jax.lax.broadcasted_iota(jnp.int32, (_BM, _N), 0)
                + b * _BM).astype(_F32)
        onehot = (rows_s[_R0:_R0 + 1, :] == rows).astype(_BF16)
        out0_ref[sl, :] = jnp.dot(onehot, z_s[...],
                                  preferred_element_type=_F32)


def _gmap(i):
    return (jnp.where(i < _T1, i, jnp.where(i >= _P8, i - _P8, 0)), 0)


def _const(shape):
    n = len(shape)
    return pl.BlockSpec(shape, lambda i: (0,) * n)


def kernel(g, h, Wd0, bd0, Wd1, bd1, Wb, bb, Wu0, bu0, Wu1, bu1,
           p0, pb0, p1, pb1):
    out0_pad, hs1, hs2 = pl.pallas_call(
        _k_mono,
        grid=(_STEPS,),
        in_specs=[pl.BlockSpec((_BM, _N), _gmap),
                  _const((_N, _D)),
                  _const((_D, _D)), _const((1, _D)),
                  _const((_D, _D)), _const((1, _D)),
                  _const((_D, _D)), _const((1, _D)),
                  _const((_D, _D)), _const((1, _D)),
                  _const((_D, _D)), _const((1, _D)),
                  _const((1, _D)), _const((1, 1)),
                  _const((1, _D)), _const((1, 1))],
        out_specs=[_const((_N, _D)), _const((_N, _D)), _const((_N, _D))],
        out_shape=[jax.ShapeDtypeStruct((_N, _D), _F32)] * 3,
        scratch_shapes=[
            pltpu.VMEM((_N, _N), _BF16),
            pltpu.VMEM((_N, _N), _BF16),
            pltpu.VMEM((_N, _D), _F32),
            pltpu.VMEM((_N, _D), _F32),
            pltpu.VMEM((_N, _D), _F32),
            pltpu.VMEM((_N, _D), _BF16),
            pltpu.VMEM((_N, _D), _BF16),
            pltpu.VMEM((_N, _D), _BF16),
            pltpu.VMEM((_N, 8), _F32),
            pltpu.VMEM((8, _N), _F32),
        ],
    )(g, h, Wd0, bd0.reshape(1, _D), Wd1, bd1.reshape(1, _D),
      Wb, bb.reshape(1, _D), Wu0, bu0.reshape(1, _D), Wu1,
      bu1.reshape(1, _D), p0.reshape(1, _D), pb0.reshape(1, 1),
      p1.reshape(1, _D), pb1.reshape(1, 1))

    return (out0_pad[:_KN0], hs1, hs2)


# speedup vs baseline: 2.7635x; 1.1190x over previous
import jax
import jax.numpy as jnp
from jax.experimental import pallas as pl
from jax.experimental.pallas import tpu as pltpu

_N = 2048
_D = 256
_KN0 = 1638
_KN1 = 982
_BM = 512
_BT = 512

_F32 = jnp.float32
_BF16 = jnp.bfloat16

_DEG, _W0, _R0, _M0, _DEG1, _W1, _M1 = 0, 1, 2, 3, 4, 5, 6

_NB = _N // _BM
_NT = _N // _BT
_P0, _T1, _P1, _T2, _P2 = 0, _NB, _NB + 1, 2 * _NB + 1, 2 * _NB + 2
_P3 = _P2 + _NT
_T3 = _P3 + _NB
_P4 = _T3 + 1
_T4 = _P4 + _NB
_P5 = _T4 + 1
_P6 = _P5 + _NT
_P7 = _P6 + _NB
_P8 = _P7 + _NB
_STEPS = _P8 + _NB


def _k_mono(g_ref, h_ref, wd0_ref, bd0_ref, wd1_ref, bd1_ref, wb_ref, bb_ref,
            wu0_ref, bu0_ref, wu1_ref, bu1_ref, p0_ref, pb0_ref, p1_ref,
            pb1_ref, out0_ref, hs1_ref, hs2_ref,
            ug_s, t_s, gbf_s, h0_s, x0_s, h1_s, x1_s, y2_s, z_s, cols_s,
            rows_s):
    i = pl.program_id(0)
    sig = jax.nn.sigmoid

    @pl.when(i < _T1)
    def _down0():
        b = i - _P0
        sl = pl.ds(b * _BM, _BM)
        gblk = g_ref[...]
        deg = jnp.maximum(jnp.sum(gblk, axis=1, keepdims=True), 1e-12)
        agg = jnp.dot(gblk, h_ref[...], preferred_element_type=_F32) / deg
        h0 = jax.nn.relu(jnp.dot(agg, wd0_ref[...],
                                 preferred_element_type=_F32) + bd0_ref[...])
        h0_s[sl, :] = h0
        cols_s[sl, _DEG:_DEG + 1] = deg
        cols_s[sl, _W0:_W0 + 1] = (jnp.sum(h0 * p0_ref[...], axis=1,
                                           keepdims=True) + pb0_ref[...])
        ug_s[sl, :] = (gblk > 0).astype(_BF16)
        gbf_s[sl, :] = gblk.astype(_BF16)

    @pl.when((i == _T1) | (i == _T2) | (i == _T3) | (i == _T4))
    def _transpose():
        rows_s[...] = jnp.transpose(cols_s[...])

    @pl.when((i >= _P1) & (i < _T2))
    def _rank0():
        b = i - _P1
        sl = pl.ds(b * _BM, _BM)
        sc = sig(cols_s[sl, _W0:_W0 + 1])
        sr = sig(rows_s[_W0:_W0 + 1, :])
        gt = jnp.sum((sr > sc).astype(_F32), axis=1, keepdims=True)
        jglob = jax.lax.broadcasted_iota(jnp.int32, (_BM, _N), 1)
        iglob = jax.lax.broadcasted_iota(jnp.int32, (_BM, _N), 0) + b * _BM
        eqb = jnp.sum(((sr == sc) & (jglob < iglob)).astype(_F32),
                      axis=1, keepdims=True)
        rank = gt + eqb
        m0 = (rank < float(_KN0)).astype(_F32)
        cols_s[sl, _R0:_R0 + 1] = rank
        cols_s[sl, _M0:_M0 + 1] = m0
        x0_s[sl, :] = h0_s[sl, :] * (m0 * sc)

    @pl.when((i >= _P2) & (i < _P3))
    def _twohop_t():
        b = i - _P2
        sl = pl.ds(b * _BT, _BT)
        cnt = jnp.dot(ug_s[sl, :], ug_s[...], preferred_element_type=_F32)
        t_s[sl, :] = (cnt > 0).astype(_BF16)

    @pl.when((i >= _P3) & (i < _T3))
    def _down1():
        b = i - _P3
        sl = pl.ds(b * _BM, _BM)
        tf = t_s[sl, :].astype(_F32)
        m0r = rows_s[_M0:_M0 + 1, :]
        deg1 = jnp.maximum(jnp.sum(tf * m0r, axis=1, keepdims=True), 1e-12)
        agg = jnp.dot(tf, x0_s[...], preferred_element_type=_F32) / deg1
        h1 = jax.nn.relu(jnp.dot(agg, wd1_ref[...],
                                 preferred_element_type=_F32) + bd1_ref[...])
        h1_s[sl, :] = h1
        cols_s[sl, _DEG1:_DEG1 + 1] = deg1
        cols_s[sl, _W1:_W1 + 1] = (jnp.sum(h1 * p1_ref[...], axis=1,
                                           keepdims=True) + pb1_ref[...])

    @pl.when((i >= _P4) & (i < _T4))
    def _rank1():
        b = i - _P4
        sl = pl.ds(b * _BM, _BM)
        sc = sig(cols_s[sl, _W1:_W1 + 1])
        sr = sig(rows_s[_W1:_W1 + 1, :])
        m0r = rows_s[_M0:_M0 + 1, :]
        r0c = cols_s[sl, _R0:_R0 + 1]
        r0r = rows_s[_R0:_R0 + 1, :]
        gt = jnp.sum((sr > sc).astype(_F32) * m0r, axis=1, keepdims=True)
        eqb = jnp.sum(((sr == sc) & (r0r < r0c)).astype(_F32) * m0r,
                      axis=1, keepdims=True)
        m1 = (gt + eqb < float(_KN1)).astype(_F32) * cols_s[sl, _M0:_M0 + 1]
        cols_s[sl, _M1:_M1 + 1] = m1
        x1_s[sl, :] = (h1_s[sl, :] * (m1 * sc)).astype(_BF16)

    @pl.when((i >= _P5) & (i < _P6))
    def _twohop_u():
        b = i - _P5
        sl = pl.ds(b * _BT, _BT)
        m0r = rows_s[_M0:_M0 + 1, :]
        a = (t_s[sl, :].astype(_F32) * m0r).astype(_BF16)
        cnt = jnp.dot(a, t_s[...], preferred_element_type=_F32)
        ug_s[sl, :] = (cnt > 0).astype(_BF16)

    @pl.when((i >= _P6) & (i < _P7))
    def _bottleneck():
        b = i - _P6
        sl = pl.ds(b * _BM, _BM)
        ub = ug_s[sl, :]
        m1r = rows_s[_M1:_M1 + 1, :]
        deg2 = jnp.maximum(jnp.sum(ub.astype(_F32) * m1r, axis=1,
                                   keepdims=True), 1e-12)
        agg = jnp.dot(ub, x1_s[...], preferred_element_type=_F32) / deg2
        h2 = jax.nn.relu(jnp.dot(agg.astype(_BF16),
                                 wb_ref[...].astype(_BF16),
                                 preferred_element_type=_F32) + bb_ref[...])
        y2_s[sl, :] = (h2 * cols_s[sl, _M1:_M1 + 1]).astype(_BF16)

    @pl.when((i >= _P7) & (i < _P8))
    def _up1():
        b = i - _P7
        sl = pl.ds(b * _BM, _BM)
        agg = (jnp.dot(t_s[sl, :], y2_s[...], preferred_element_type=_F32)
               / cols_s[sl, _DEG1:_DEG1 + 1])
        hu = jax.nn.relu(jnp.dot(agg.astype(_BF16),
                                 wu0_ref[...].astype(_BF16),
                                 preferred_element_type=_F32) + bu0_ref[...])
        z_s[sl, :] = ((hu + h1_s[sl, :])
                      * cols_s[sl, _M0:_M0 + 1]).astype(_BF16)

    @pl.when(i >= _P8)
    def _up0_compact():
        b = i - _P8
        sl = pl.ds(b * _BM, _BM)
        agg = (jnp.dot(gbf_s[sl, :], z_s[...], preferred_element_type=_F32)
               / cols_s[sl, _DEG:_DEG + 1])
        hu = jax.nn.relu(jnp.dot(agg.astype(_BF16),
                                 wu1_ref[...].astype(_BF16),
                                 preferred_element_type=_F32) + bu1_ref[...])
        hs1 = hu + h0_s[sl, :]
        hs1_ref[sl, :] = hs1
        hs2_ref[sl, :] = hs1 + h_ref[sl, :]
        rows = (jax.lax.broadcasted_iota(jnp.int32, (_BM, _N), 0)
                + b * _BM).astype(_F32)
        onehot = (rows_s[_R0:_R0 + 1, :] == rows).astype(_BF16)
        out0_ref[sl, :] = jnp.dot(onehot, z_s[...],
                                  preferred_element_type=_F32)


def _gmap(i):
    return (jnp.where(i < _T1, i, 0), 0)


def _const(shape):
    n = len(shape)
    return pl.BlockSpec(shape, lambda i: (0,) * n)


def kernel(g, h, Wd0, bd0, Wd1, bd1, Wb, bb, Wu0, bu0, Wu1, bu1,
           p0, pb0, p1, pb1):
    out0_pad, hs1, hs2 = pl.pallas_call(
        _k_mono,
        grid=(_STEPS,),
        in_specs=[pl.BlockSpec((_BM, _N), _gmap),
                  _const((_N, _D)),
                  _const((_D, _D)), _const((1, _D)),
                  _const((_D, _D)), _const((1, _D)),
                  _const((_D, _D)), _const((1, _D)),
                  _const((_D, _D)), _const((1, _D)),
                  _const((_D, _D)), _const((1, _D)),
                  _const((1, _D)), _const((1, 1)),
                  _const((1, _D)), _const((1, 1))],
        out_specs=[_const((_N, _D)), _const((_N, _D)), _const((_N, _D))],
        out_shape=[jax.ShapeDtypeStruct((_N, _D), _F32)] * 3,
        scratch_shapes=[
            pltpu.VMEM((_N, _N), _BF16),
            pltpu.VMEM((_N, _N), _BF16),
            pltpu.VMEM((_N, _N), _BF16),
            pltpu.VMEM((_N, _D), _F32),
            pltpu.VMEM((_N, _D), _F32),
            pltpu.VMEM((_N, _D), _F32),
            pltpu.VMEM((_N, _D), _BF16),
            pltpu.VMEM((_N, _D), _BF16),
            pltpu.VMEM((_N, _D), _BF16),
            pltpu.VMEM((_N, 8), _F32),
            pltpu.VMEM((8, _N), _F32),
        ],
    )(g, h, Wd0, bd0.reshape(1, _D), Wd1, bd1.reshape(1, _D),
      Wb, bb.reshape(1, _D), Wu0, bu0.reshape(1, _D), Wu1,
      bu1.reshape(1, _D), p0.reshape(1, _D), pb0.reshape(1, 1),
      p1.reshape(1, _D), pb1.reshape(1, 1))

    return (out0_pad[:_KN0], hs1, hs2)


# speedup vs baseline: 2.8022x; 1.0140x over previous
import jax
import jax.numpy as jnp
from jax.experimental import pallas as pl
from jax.experimental.pallas import tpu as pltpu

_N = 2048
_D = 256
_KN0 = 1638
_KN1 = 982
_BM = 512
_BT = 1024

_F32 = jnp.float32
_BF16 = jnp.bfloat16

_DEG, _W0, _R0, _M0, _DEG1, _W1, _M1 = 0, 1, 2, 3, 4, 5, 6

_NB = _N // _BM
_NT = _N // _BT
_P0, _T1, _P1, _T2, _P2 = 0, _NB, _NB + 1, 2 * _NB + 1, 2 * _NB + 2
_P3 = _P2 + _NT
_T3 = _P3 + _NB
_P4 = _T3 + 1
_T4 = _P4 + _NB
_P5 = _T4 + 1
_P6 = _P5 + _NT
_P7 = _P6 + _NB
_P8 = _P7 + _NB
_STEPS = _P8 + _NB


def _k_mono(g_ref, h_ref, wd0_ref, bd0_ref, wd1_ref, bd1_ref, wb_ref, bb_ref,
            wu0_ref, bu0_ref, wu1_ref, bu1_ref, p0_ref, pb0_ref, p1_ref,
            pb1_ref, out0_ref, hs1_ref, hs2_ref,
            ug_s, t_s, gbf_s, h0_s, x0_s, h1_s, x1_s, y2_s, z_s, cols_s,
            rows_s):
    i = pl.program_id(0)
    sig = jax.nn.sigmoid

    @pl.when(i < _T1)
    def _down0():
        b = i - _P0
        sl = pl.ds(b * _BM, _BM)
        gblk = g_ref[...]
        deg = jnp.maximum(jnp.sum(gblk, axis=1, keepdims=True), 1e-12)
        agg = jnp.dot(gblk, h_ref[...], preferred_element_type=_F32) / deg
        h0 = jax.nn.relu(jnp.dot(agg, wd0_ref[...],
                                 preferred_element_type=_F32) + bd0_ref[...])
        h0_s[sl, :] = h0
        cols_s[sl, _DEG:_DEG + 1] = deg
        cols_s[sl, _W0:_W0 + 1] = (jnp.sum(h0 * p0_ref[...], axis=1,
                                           keepdims=True) + pb0_ref[...])
        ug_s[sl, :] = (gblk > 0).astype(_BF16)
        gbf_s[sl, :] = gblk.astype(_BF16)

    @pl.when((i == _T1) | (i == _T2) | (i == _T3) | (i == _T4))
    def _transpose():
        rows_s[...] = jnp.transpose(cols_s[...])

    @pl.when((i >= _P1) & (i < _T2))
    def _rank0():
        b = i - _P1
        sl = pl.ds(b * _BM, _BM)
        sc = sig(cols_s[sl, _W0:_W0 + 1])
        sr = sig(rows_s[_W0:_W0 + 1, :])
        gt = jnp.sum((sr > sc).astype(_F32), axis=1, keepdims=True)
        jglob = jax.lax.broadcasted_iota(jnp.int32, (_BM, _N), 1)
        iglob = jax.lax.broadcasted_iota(jnp.int32, (_BM, _N), 0) + b * _BM
        eqb = jnp.sum(((sr == sc) & (jglob < iglob)).astype(_F32),
                      axis=1, keepdims=True)
        rank = gt + eqb
        m0 = (rank < float(_KN0)).astype(_F32)
        cols_s[sl, _R0:_R0 + 1] = rank
        cols_s[sl, _M0:_M0 + 1] = m0
        x0_s[sl, :] = h0_s[sl, :] * (m0 * sc)

    @pl.when((i >= _P2) & (i < _P3))
    def _twohop_t():
        b = i - _P2
        sl = pl.ds(b * _BT, _BT)
        cnt = jnp.dot(ug_s[sl, :], ug_s[...], preferred_element_type=_F32)
        t_s[sl, :] = (cnt > 0).astype(_BF16)

    @pl.when((i >= _P3) & (i < _T3))
    def _down1():
        b = i - _P3
        sl = pl.ds(b * _BM, _BM)
        tf = t_s[sl, :].astype(_F32)
        m0r = rows_s[_M0:_M0 + 1, :]
        deg1 = jnp.maximum(jnp.sum(tf * m0r, axis=1, keepdims=True), 1e-12)
        agg = jnp.dot(tf, x0_s[...], preferred_element_type=_F32) / deg1
        h1 = jax.nn.relu(jnp.dot(agg, wd1_ref[...],
                                 preferred_element_type=_F32) + bd1_ref[...])
        h1_s[sl, :] = h1
        cols_s[sl, _DEG1:_DEG1 + 1] = deg1
        cols_s[sl, _W1:_W1 + 1] = (jnp.sum(h1 * p1_ref[...], axis=1,
                                           keepdims=True) + pb1_ref[...])

    @pl.when((i >= _P4) & (i < _T4))
    def _rank1():
        b = i - _P4
        sl = pl.ds(b * _BM, _BM)
        sc = sig(cols_s[sl, _W1:_W1 + 1])
        sr = sig(rows_s[_W1:_W1 + 1, :])
        m0r = rows_s[_M0:_M0 + 1, :]
        r0c = cols_s[sl, _R0:_R0 + 1]
        r0r = rows_s[_R0:_R0 + 1, :]
        gt = jnp.sum((sr > sc).astype(_F32) * m0r, axis=1, keepdims=True)
        eqb = jnp.sum(((sr == sc) & (r0r < r0c)).astype(_F32) * m0r,
                      axis=1, keepdims=True)
        m1 = (gt + eqb < float(_KN1)).astype(_F32) * cols_s[sl, _M0:_M0 + 1]
        cols_s[sl, _M1:_M1 + 1] = m1
        x1_s[sl, :] = (h1_s[sl, :] * (m1 * sc)).astype(_BF16)

    @pl.when((i >= _P5) & (i < _P6))
    def _twohop_u():
        b = i - _P5
        sl = pl.ds(b * _BT, _BT)
        m0r = rows_s[_M0:_M0 + 1, :].astype(_BF16)
        a = t_s[sl, :] * m0r
        cnt = jnp.dot(a, t_s[...], preferred_element_type=_F32)
        ug_s[sl, :] = (cnt > 0).astype(_BF16)

    @pl.when((i >= _P6) & (i < _P7))
    def _bottleneck():
        b = i - _P6
        sl = pl.ds(b * _BM, _BM)
        ub = ug_s[sl, :]
        m1r = rows_s[_M1:_M1 + 1, :]
        deg2 = jnp.maximum(jnp.sum(ub.astype(_F32) * m1r, axis=1,
                                   keepdims=True), 1e-12)
        agg = jnp.dot(ub, x1_s[...], preferred_element_type=_F32) / deg2
        h2 = jax.nn.relu(jnp.dot(agg.astype(_BF16),
                                 wb_ref[...].astype(_BF16),
                                 preferred_element_type=_F32) + bb_ref[...])
        y2_s[sl, :] = (h2 * cols_s[sl, _M1:_M1 + 1]).astype(_BF16)

    @pl.when((i >= _P7) & (i < _P8))
    def _up1():
        b = i - _P7
        sl = pl.ds(b * _BM, _BM)
        agg = (jnp.dot(t_s[sl, :], y2_s[...], preferred_element_type=_F32)
               / cols_s[sl, _DEG1:_DEG1 + 1])
        hu = jax.nn.relu(jnp.dot(agg.astype(_BF16),
                                 wu0_ref[...].astype(_BF16),
                                 preferred_element_type=_F32) + bu0_ref[...])
        z_s[sl, :] = ((hu + h1_s[sl, :])
                      * cols_s[sl, _M0:_M0 + 1]).astype(_BF16)

    @pl.when(i >= _P8)
    def _up0_compact():
        b = i - _P8
        sl = pl.ds(b * _BM, _BM)
        agg = (jnp.dot(gbf_s[sl, :], z_s[...], preferred_element_type=_F32)
               / cols_s[sl, _DEG:_DEG + 1])
        hu = jax.nn.relu(jnp.dot(agg.astype(_BF16),
                                 wu1_ref[...].astype(_BF16),
                                 preferred_element_type=_F32) + bu1_ref[...])
        hs1 = hu + h0_s[sl, :]
        hs1_ref[sl, :] = hs1
        hs2_ref[sl, :] = hs1 + h_ref[sl, :]
        rows = (jax.lax.broadcasted_iota(jnp.int32, (_BM, _N), 0)
                + b * _BM).astype(_F32)
        onehot = (rows_s[_R0:_R0 + 1, :] == rows).astype(_BF16)
        out0_ref[sl, :] = jnp.dot(onehot, z_s[...],
                                  preferred_element_type=_F32)


def _gmap(i):
    return (jnp.where(i < _T1, i, 0), 0)


def _const(shape):
    n = len(shape)
    return pl.BlockSpec(shape, lambda i: (0,) * n)


def kernel(g, h, Wd0, bd0, Wd1, bd1, Wb, bb, Wu0, bu0, Wu1, bu1,
           p0, pb0, p1, pb1):
    out0_pad, hs1, hs2 = pl.pallas_call(
        _k_mono,
        grid=(_STEPS,),
        in_specs=[pl.BlockSpec((_BM, _N), _gmap),
                  _const((_N, _D)),
                  _const((_D, _D)), _const((1, _D)),
                  _const((_D, _D)), _const((1, _D)),
                  _const((_D, _D)), _const((1, _D)),
                  _const((_D, _D)), _const((1, _D)),
                  _const((_D, _D)), _const((1, _D)),
                  _const((1, _D)), _const((1, 1)),
                  _const((1, _D)), _const((1, 1))],
        out_specs=[_const((_N, _D)), _const((_N, _D)), _const((_N, _D))],
        out_shape=[jax.ShapeDtypeStruct((_N, _D), _F32)] * 3,
        scratch_shapes=[
            pltpu.VMEM((_N, _N), _BF16),
            pltpu.VMEM((_N, _N), _BF16),
            pltpu.VMEM((_N, _N), _BF16),
            pltpu.VMEM((_N, _D), _F32),
            pltpu.VMEM((_N, _D), _F32),
            pltpu.VMEM((_N, _D), _F32),
            pltpu.VMEM((_N, _D), _BF16),
            pltpu.VMEM((_N, _D), _BF16),
            pltpu.VMEM((_N, _D), _BF16),
            pltpu.VMEM((_N, 8), _F32),
            pltpu.VMEM((8, _N), _F32),
        ],
    )(g, h, Wd0, bd0.reshape(1, _D), Wd1, bd1.reshape(1, _D),
      Wb, bb.reshape(1, _D), Wu0, bu0.reshape(1, _D), Wu1,
      bu1.reshape(1, _D), p0.reshape(1, _D), pb0.reshape(1, 1),
      p1.reshape(1, _D), pb1.reshape(1, 1))

    return (out0_pad[:_KN0], hs1, hs2)


# speedup vs baseline: 2.9427x; 1.0501x over previous
import jax
import jax.numpy as jnp
from jax.experimental import pallas as pl
from jax.experimental.pallas import tpu as pltpu

_N = 2048
_D = 256
_KN0 = 1638
_KN1 = 982
_BM = 512
_BT = 512

_F32 = jnp.float32
_BF16 = jnp.bfloat16

_DEG, _W0, _R0, _M0, _DEG1, _W1, _M1 = 0, 1, 2, 3, 4, 5, 6

_NB = _N // _BM
_P0 = 0
_T1 = _P0 + _NB
_PA = _T1 + 1
_T2 = _PA + _NB
_P3 = _T2 + 1
_T3 = _P3 + _NB
_PB = _T3 + 1
_T4 = _PB + _NB
_P6 = _T4 + 1
_P7 = _P6 + _NB
_P8 = _P7 + _NB
_STEPS = _P8 + _NB


def _k_mono(g_ref, h_ref, wd0_ref, bd0_ref, wd1_ref, bd1_ref, wb_ref, bb_ref,
            wu0_ref, bu0_ref, wu1_ref, bu1_ref, p0_ref, pb0_ref, p1_ref,
            pb1_ref, out0_ref, hs1_ref, hs2_ref,
            ug_s, t_s, gbf_s, h0_s, x0_s, h1_s, x1_s, y2_s, z_s, cols_s,
            rows_s):
    i = pl.program_id(0)
    sig = jax.nn.sigmoid

    @pl.when(i < _T1)
    def _down0():
        b = i - _P0
        sl = pl.ds(b * _BM, _BM)
        gblk = g_ref[...]
        deg = jnp.maximum(jnp.sum(gblk, axis=1, keepdims=True), 1e-12)
        agg = jnp.dot(gblk, h_ref[...], preferred_element_type=_F32) / deg
        h0 = jax.nn.relu(jnp.dot(agg, wd0_ref[...],
                                 preferred_element_type=_F32) + bd0_ref[...])
        h0_s[sl, :] = h0
        cols_s[sl, _DEG:_DEG + 1] = deg
        cols_s[sl, _W0:_W0 + 1] = (jnp.sum(h0 * p0_ref[...], axis=1,
                                           keepdims=True) + pb0_ref[...])
        ug_s[sl, :] = (gblk > 0).astype(_BF16)
        gbf_s[sl, :] = gblk.astype(_BF16)

    @pl.when((i == _T1) | (i == _T2) | (i == _T3) | (i == _T4))
    def _transpose():
        rows_s[...] = jnp.transpose(cols_s[...])

    @pl.when((i >= _PA) & (i < _T2))
    def _twohop_t_rank0():
        b = i - _PA
        tsl = pl.ds(b * _BT, _BT)
        cnt = jnp.dot(ug_s[tsl, :], ug_s[...], preferred_element_type=_F32)
        t_s[tsl, :] = (cnt > 0).astype(_BF16)
        sl = pl.ds(b * _BM, _BM)
        sc = sig(cols_s[sl, _W0:_W0 + 1])
        sr = sig(rows_s[_W0:_W0 + 1, :])
        gt = jnp.sum((sr > sc).astype(_F32), axis=1, keepdims=True)
        jglob = jax.lax.broadcasted_iota(jnp.int32, (_BM, _N), 1)
        iglob = jax.lax.broadcasted_iota(jnp.int32, (_BM, _N), 0) + b * _BM
        eqb = jnp.sum(((sr == sc) & (jglob < iglob)).astype(_F32),
                      axis=1, keepdims=True)
        rank = gt + eqb
        m0 = (rank < float(_KN0)).astype(_F32)
        cols_s[sl, _R0:_R0 + 1] = rank
        cols_s[sl, _M0:_M0 + 1] = m0
        x0_s[sl, :] = h0_s[sl, :] * (m0 * sc)

    @pl.when((i >= _P3) & (i < _T3))
    def _down1():
        b = i - _P3
        sl = pl.ds(b * _BM, _BM)
        tf = t_s[sl, :].astype(_F32)
        m0r = rows_s[_M0:_M0 + 1, :]
        deg1 = jnp.maximum(jnp.sum(tf * m0r, axis=1, keepdims=True), 1e-12)
        agg = jnp.dot(tf, x0_s[...], preferred_element_type=_F32) / deg1
        h1 = jax.nn.relu(jnp.dot(agg, wd1_ref[...],
                                 preferred_element_type=_F32) + bd1_ref[...])
        h1_s[sl, :] = h1
        cols_s[sl, _DEG1:_DEG1 + 1] = deg1
        cols_s[sl, _W1:_W1 + 1] = (jnp.sum(h1 * p1_ref[...], axis=1,
                                           keepdims=True) + pb1_ref[...])

    @pl.when((i >= _PB) & (i < _T4))
    def _twohop_u_rank1():
        b = i - _PB
        tsl = pl.ds(b * _BT, _BT)
        m0rb = rows_s[_M0:_M0 + 1, :].astype(_BF16)
        a = t_s[tsl, :] * m0rb
        cnt = jnp.dot(a, t_s[...], preferred_element_type=_F32)
        ug_s[tsl, :] = (cnt > 0).astype(_BF16)
        sl = pl.ds(b * _BM, _BM)
        sc = sig(cols_s[sl, _W1:_W1 + 1])
        sr = sig(rows_s[_W1:_W1 + 1, :])
        m0r = rows_s[_M0:_M0 + 1, :]
        r0c = cols_s[sl, _R0:_R0 + 1]
        r0r = rows_s[_R0:_R0 + 1, :]
        gt = jnp.sum((sr > sc).astype(_F32) * m0r, axis=1, keepdims=True)
        eqb = jnp.sum(((sr == sc) & (r0r < r0c)).astype(_F32) * m0r,
                      axis=1, keepdims=True)
        m1 = (gt + eqb < float(_KN1)).astype(_F32) * cols_s[sl, _M0:_M0 + 1]
        cols_s[sl, _M1:_M1 + 1] = m1
        x1_s[sl, :] = (h1_s[sl, :] * (m1 * sc)).astype(_BF16)

    @pl.when((i >= _P6) & (i < _P7))
    def _bottleneck():
        b = i - _P6
        sl = pl.ds(b * _BM, _BM)
        ub = ug_s[sl, :]
        m1r = rows_s[_M1:_M1 + 1, :]
        deg2 = jnp.maximum(jnp.sum(ub.astype(_F32) * m1r, axis=1,
                                   keepdims=True), 1e-12)
        agg = jnp.dot(ub, x1_s[...], preferred_element_type=_F32) / deg2
        h2 = jax.nn.relu(jnp.dot(agg.astype(_BF16),
                                 wb_ref[...].astype(_BF16),
                                 preferred_element_type=_F32) + bb_ref[...])
        y2_s[sl, :] = (h2 * cols_s[sl, _M1:_M1 + 1]).astype(_BF16)

    @pl.when((i >= _P7) & (i < _P8))
    def _up1():
        b = i - _P7
        sl = pl.ds(b * _BM, _BM)
        agg = (jnp.dot(t_s[sl, :], y2_s[...], preferred_element_type=_F32)
               / cols_s[sl, _DEG1:_DEG1 + 1])
        hu = jax.nn.relu(jnp.dot(agg.astype(_BF16),
                                 wu0_ref[...].astype(_BF16),
                                 preferred_element_type=_F32) + bu0_ref[...])
        z_s[sl, :] = ((hu + h1_s[sl, :])
                      * cols_s[sl, _M0:_M0 + 1]).astype(_BF16)

    @pl.when(i >= _P8)
    def _up0_compact():
        b = i - _P8
        sl = pl.ds(b * _BM, _BM)
        agg = (jnp.dot(gbf_s[sl, :], z_s[...], preferred_element_type=_F32)
               / cols_s[sl, _DEG:_DEG + 1])
        hu = jax.nn.relu(jnp.dot(agg.astype(_BF16),
                                 wu1_ref[...].astype(_BF16),
                                 preferred_element_type=_F32) + bu1_ref[...])
        hs1 = hu + h0_s[sl, :]
        hs1_ref[sl, :] = hs1
        hs2_ref[sl, :] = hs1 + h_ref[sl, :]
        rows = (jax.lax.broadcasted_iota(jnp.int32, (_BM, _N), 0)
                + b * _BM).astype(_F32)
        onehot = (rows_s[_R0:_R0 + 1, :] == rows).astype(_BF16)
        out0_ref[sl, :] = jnp.dot(onehot, z_s[...],
                                  preferred_element_type=_F32)


def _gmap(i):
    return (jnp.where(i < _T1, i, 0), 0)


def _const(shape):
    n = len(shape)
    return pl.BlockSpec(shape, lambda i: (0,) * n)


def kernel(g, h, Wd0, bd0, Wd1, bd1, Wb, bb, Wu0, bu0, Wu1, bu1,
           p0, pb0, p1, pb1):
    out0_pad, hs1, hs2 = pl.pallas_call(
        _k_mono,
        grid=(_STEPS,),
        in_specs=[pl.BlockSpec((_BM, _N), _gmap),
                  _const((_N, _D)),
                  _const((_D, _D)), _const((1, _D)),
                  _const((_D, _D)), _const((1, _D)),
                  _const((_D, _D)), _const((1, _D)),
                  _const((_D, _D)), _const((1, _D)),
                  _const((_D, _D)), _const((1, _D)),
                  _const((1, _D)), _const((1, 1)),
                  _const((1, _D)), _const((1, 1))],
        out_specs=[_const((_N, _D)), _const((_N, _D)), _const((_N, _D))],
        out_shape=[jax.ShapeDtypeStruct((_N, _D), _F32)] * 3,
        scratch_shapes=[
            pltpu.VMEM((_N, _N), _BF16),
            pltpu.VMEM((_N, _N), _BF16),
            pltpu.VMEM((_N, _N), _BF16),
            pltpu.VMEM((_N, _D), _F32),
            pltpu.VMEM((_N, _D), _F32),
            pltpu.VMEM((_N, _D), _F32),
            pltpu.VMEM((_N, _D), _BF16),
            pltpu.VMEM((_N, _D), _BF16),
            pltpu.VMEM((_N, _D), _BF16),
            pltpu.VMEM((_N, 8), _F32),
            pltpu.VMEM((8, _N), _F32),
        ],
    )(g, h, Wd0, bd0.reshape(1, _D), Wd1, bd1.reshape(1, _D),
      Wb, bb.reshape(1, _D), Wu0, bu0.reshape(1, _D), Wu1,
      bu1.reshape(1, _D), p0.reshape(1, _D), pb0.reshape(1, 1),
      p1.reshape(1, _D), pb1.reshape(1, 1))

    return (out0_pad[:_KN0], hs1, hs2)


# speedup vs baseline: 2.9528x; 1.0034x over previous
import jax
import jax.numpy as jnp
from jax.experimental import pallas as pl
from jax.experimental.pallas import tpu as pltpu

_N = 2048
_D = 256
_KN0 = 1638
_KN1 = 982
_BM = 512
_BT = 512

_F32 = jnp.float32
_BF16 = jnp.bfloat16

_DEG, _W0, _R0, _M0, _DEG1, _W1, _M1 = 0, 1, 2, 3, 4, 5, 6

_NB = _N // _BM
_P0 = 0
_PA = _P0 + _NB
_P3 = _PA + _NB
_PB = _P3 + _NB
_P6 = _PB + _NB
_P7 = _P6 + _NB
_P8 = _P7 + _NB
_STEPS = _P8 + _NB


def _k_mono(g_ref, h_ref, wd0_ref, bd0_ref, wd1_ref, bd1_ref, wb_ref, bb_ref,
            wu0_ref, bu0_ref, wu1_ref, bu1_ref, p0_ref, pb0_ref, p1_ref,
            pb1_ref, out0_ref, hs1_ref, hs2_ref,
            ug_s, t_s, gbf_s, h0_s, x0_s, h1_s, x1_s, y2_s, z_s, cols_s,
            rows_s):
    i = pl.program_id(0)
    sig = jax.nn.sigmoid

    @pl.when(i < _PA)
    def _down0():
        b = i - _P0
        sl = pl.ds(b * _BM, _BM)
        gblk = g_ref[...]
        deg = jnp.maximum(jnp.sum(gblk, axis=1, keepdims=True), 1e-12)
        agg = jnp.dot(gblk, h_ref[...], preferred_element_type=_F32) / deg
        h0 = jax.nn.relu(jnp.dot(agg, wd0_ref[...],
                                 preferred_element_type=_F32) + bd0_ref[...])
        h0_s[sl, :] = h0
        cols_s[sl, _DEG:_DEG + 1] = deg
        cols_s[sl, _W0:_W0 + 1] = (jnp.sum(h0 * p0_ref[...], axis=1,
                                           keepdims=True) + pb0_ref[...])
        ug_s[sl, :] = (gblk > 0).astype(_BF16)
        gbf_s[sl, :] = gblk.astype(_BF16)

    @pl.when((i == _PA) | (i == _P3) | (i == _PB) | (i == _P6))
    def _transpose():
        rows_s[...] = jnp.transpose(cols_s[...])

    @pl.when((i >= _PA) & (i < _P3))
    def _twohop_t_rank0():
        b = i - _PA
        tsl = pl.ds(b * _BT, _BT)
        cnt = jnp.dot(ug_s[tsl, :], ug_s[...], preferred_element_type=_F32)
        t_s[tsl, :] = (cnt > 0).astype(_BF16)
        sl = pl.ds(b * _BM, _BM)
        sc = sig(cols_s[sl, _W0:_W0 + 1])
        sr = sig(rows_s[_W0:_W0 + 1, :])
        gt = jnp.sum((sr > sc).astype(_F32), axis=1, keepdims=True)
        jglob = jax.lax.broadcasted_iota(jnp.int32, (_BM, _N), 1)
        iglob = jax.lax.broadcasted_iota(jnp.int32, (_BM, _N), 0) + b * _BM
        eqb = jnp.sum(((sr == sc) & (jglob < iglob)).astype(_F32),
                      axis=1, keepdims=True)
        rank = gt + eqb
        m0 = (rank < float(_KN0)).astype(_F32)
        cols_s[sl, _R0:_R0 + 1] = rank
        cols_s[sl, _M0:_M0 + 1] = m0
        x0_s[sl, :] = h0_s[sl, :] * (m0 * sc)

    @pl.when((i >= _P3) & (i < _PB))
    def _down1():
        b = i - _P3
        sl = pl.ds(b * _BM, _BM)
        tf = t_s[sl, :].astype(_F32)
        m0r = rows_s[_M0:_M0 + 1, :]
        deg1 = jnp.maximum(jnp.sum(tf * m0r, axis=1, keepdims=True), 1e-12)
        agg = jnp.dot(tf, x0_s[...], preferred_element_type=_F32) / deg1
        h1 = jax.nn.relu(jnp.dot(agg, wd1_ref[...],
                                 preferred_element_type=_F32) + bd1_ref[...])
        h1_s[sl, :] = h1
        cols_s[sl, _DEG1:_DEG1 + 1] = deg1
        cols_s[sl, _W1:_W1 + 1] = (jnp.sum(h1 * p1_ref[...], axis=1,
                                           keepdims=True) + pb1_ref[...])

    @pl.when((i >= _PB) & (i < _P6))
    def _twohop_u_rank1():
        b = i - _PB
        tsl = pl.ds(b * _BT, _BT)
        m0rb = rows_s[_M0:_M0 + 1, :].astype(_BF16)
        a = t_s[tsl, :] * m0rb
        cnt = jnp.dot(a, t_s[...], preferred_element_type=_F32)
        ug_s[tsl, :] = (cnt > 0).astype(_BF16)
        sl = pl.ds(b * _BM, _BM)
        sc = sig(cols_s[sl, _W1:_W1 + 1])
        sr = sig(rows_s[_W1:_W1 + 1, :])
        m0r = rows_s[_M0:_M0 + 1, :]
        r0c = cols_s[sl, _R0:_R0 + 1]
        r0r = rows_s[_R0:_R0 + 1, :]
        gt = jnp.sum((sr > sc).astype(_F32) * m0r, axis=1, keepdims=True)
        eqb = jnp.sum(((sr == sc) & (r0r < r0c)).astype(_F32) * m0r,
                      axis=1, keepdims=True)
        m1 = (gt + eqb < float(_KN1)).astype(_F32) * cols_s[sl, _M0:_M0 + 1]
        cols_s[sl, _M1:_M1 + 1] = m1
        x1_s[sl, :] = (h1_s[sl, :] * (m1 * sc)).astype(_BF16)

    @pl.when((i >= _P6) & (i < _P7))
    def _bottleneck():
        b = i - _P6
        sl = pl.ds(b * _BM, _BM)
        ub = ug_s[sl, :]
        m1r = rows_s[_M1:_M1 + 1, :]
        deg2 = jnp.maximum(jnp.sum(ub.astype(_F32) * m1r, axis=1,
                                   keepdims=True), 1e-12)
        agg = jnp.dot(ub, x1_s[...], preferred_element_type=_F32) / deg2
        h2 = jax.nn.relu(jnp.dot(agg.astype(_BF16),
                                 wb_ref[...].astype(_BF16),
                                 preferred_element_type=_F32) + bb_ref[...])
        y2_s[sl, :] = (h2 * cols_s[sl, _M1:_M1 + 1]).astype(_BF16)

    @pl.when((i >= _P7) & (i < _P8))
    def _up1():
        b = i - _P7
        sl = pl.ds(b * _BM, _BM)
        agg = (jnp.dot(t_s[sl, :], y2_s[...], preferred_element_type=_F32)
               / cols_s[sl, _DEG1:_DEG1 + 1])
        hu = jax.nn.relu(jnp.dot(agg.astype(_BF16),
                                 wu0_ref[...].astype(_BF16),
                                 preferred_element_type=_F32) + bu0_ref[...])
        z_s[sl, :] = ((hu + h1_s[sl, :])
                      * cols_s[sl, _M0:_M0 + 1]).astype(_BF16)

    @pl.when(i >= _P8)
    def _up0_compact():
        b = i - _P8
        sl = pl.ds(b * _BM, _BM)
        agg = (jnp.dot(gbf_s[sl, :], z_s[...], preferred_element_type=_F32)
               / cols_s[sl, _DEG:_DEG + 1])
        hu = jax.nn.relu(jnp.dot(agg.astype(_BF16),
                                 wu1_ref[...].astype(_BF16),
                                 preferred_element_type=_F32) + bu1_ref[...])
        hs1 = hu + h0_s[sl, :]
        hs1_ref[sl, :] = hs1
        hs2_ref[sl, :] = hs1 + h_ref[sl, :]
        rows = (jax.lax.broadcasted_iota(jnp.int32, (_BM, _N), 0)
                + b * _BM).astype(_F32)
        onehot = (rows_s[_R0:_R0 + 1, :] == rows).astype(_BF16)
        out0_ref[sl, :] = jnp.dot(onehot, z_s[...],
                                  preferred_element_type=_F32)


def _gmap(i):
    return (jnp.where(i < _PA, i, 0), 0)


def _const(shape):
    n = len(shape)
    return pl.BlockSpec(shape, lambda i: (0,) * n)


def kernel(g, h, Wd0, bd0, Wd1, bd1, Wb, bb, Wu0, bu0, Wu1, bu1,
           p0, pb0, p1, pb1):
    out0_pad, hs1, hs2 = pl.pallas_call(
        _k_mono,
        grid=(_STEPS,),
        in_specs=[pl.BlockSpec((_BM, _N), _gmap),
                  _const((_N, _D)),
                  _const((_D, _D)), _const((1, _D)),
                  _const((_D, _D)), _const((1, _D)),
                  _const((_D, _D)), _const((1, _D)),
                  _const((_D, _D)), _const((1, _D)),
                  _const((_D, _D)), _const((1, _D)),
                  _const((1, _D)), _const((1, 1)),
                  _const((1, _D)), _const((1, 1))],
        out_specs=[_const((_N, _D)), _const((_N, _D)), _const((_N, _D))],
        out_shape=[jax.ShapeDtypeStruct((_N, _D), _F32)] * 3,
        scratch_shapes=[
            pltpu.VMEM((_N, _N), _BF16),
            pltpu.VMEM((_N, _N), _BF16),
            pltpu.VMEM((_N, _N), _BF16),
            pltpu.VMEM((_N, _D), _F32),
            pltpu.VMEM((_N, _D), _F32),
            pltpu.VMEM((_N, _D), _F32),
            pltpu.VMEM((_N, _D), _BF16),
            pltpu.VMEM((_N, _D), _BF16),
            pltpu.VMEM((_N, _D), _BF16),
            pltpu.VMEM((_N, 8), _F32),
            pltpu.VMEM((8, _N), _F32),
        ],
    )(g, h, Wd0, bd0.reshape(1, _D), Wd1, bd1.reshape(1, _D),
      Wb, bb.reshape(1, _D), Wu0, bu0.reshape(1, _D), Wu1,
      bu1.reshape(1, _D), p0.reshape(1, _D), pb0.reshape(1, 1),
      p1.reshape(1, _D), pb1.reshape(1, 1))

    return (out0_pad[:_KN0], hs1, hs2)
